# baseline (speedup 1.0000x reference)
import functools

import jax
import jax.numpy as jnp
from jax import lax
from jax.experimental import pallas as pl
from jax.experimental.pallas import tpu as pltpu
from jax.experimental.pallas import tpu_sc as plsc

P = 256
K = 8
LIM = 5

_neg_idx_cache = {}


def _neg_indices(n):
    if n not in _neg_idx_cache:
        kn = jax.random.key(42)
        _neg_idx_cache[n] = jax.random.choice(kn, n, shape=(P,), replace=False)
    return _neg_idx_cache[n]


def _make_sc_gather(cf, n):
    chalf = cf // 2
    mesh = plsc.VectorSubcoreMesh(core_axis_name="c", subcore_axis_name="s")

    @functools.partial(
        pl.kernel, mesh=mesh,
        out_type=jax.ShapeDtypeStruct((2, 16, chalf * 16), jnp.float32),
        scratch_types=[
            pltpu.VMEM((16,), jnp.int32),
            pltpu.VMEM((chalf * 16,), jnp.int32),
            pltpu.VMEM((chalf * 16,), jnp.float32),
            pltpu.SemaphoreType.DMA,
        ],
    )
    def _gather(f1flat_hbm, idx_hbm, out_hbm, idx_v, flatidx_v, vals_v, sem):
        g = lax.axis_index("s")
        co = lax.axis_index("c")
        pltpu.sync_copy(idx_hbm.at[pl.ds(g * 16, 16)], idx_v)
        idx16 = idx_v[...]
        for cc in range(chalf):
            flatidx_v[pl.ds(cc * 16, 16)] = (co * chalf + cc) * n + idx16
        pltpu.async_copy(f1flat_hbm.at[flatidx_v], vals_v, sem).wait()
        pltpu.sync_copy(vals_v, out_hbm.at[co, g])

    return _gather


def _dist_topk_body(idx_ref, a_ref, f2_ref, out_ref, topk_ref, cand_ref,
                    *, chunk, nsteps, w):
    half = pl.program_id(0)
    step = pl.program_id(1)

    @pl.when(step == 0)
    def _init():
        topk_ref[...] = jnp.full((P, K), jnp.inf, jnp.float32)

    a = a_ref[...]
    a2 = jnp.sum(a * a, axis=1, keepdims=True)
    rows = chunk // w
    idx = idx_ref[...]
    gstep = half * nsteps + step

    parts = []
    for r in range(rows):
        f2r = f2_ref[:, r, :]
        b2 = jnp.sum(f2r * f2r, axis=0, keepdims=True)
        ab = jnp.dot(a, f2r, preferred_element_type=jnp.float32,
                     precision=jax.lax.Precision.DEFAULT)
        d = jax.nn.relu(a2 + b2 - 2.0 * ab)
        j = (gstep * chunk + r * w
             + jax.lax.broadcasted_iota(jnp.int32, (P, w), 1))
        q = j - idx + (w * LIM + LIM)
        excl = (j > 0) & (q >= 0) & (q < w * 2 * LIM) & ((q & (w - 1)) < 2 * LIM)
        parts.append(jnp.where(excl, d + 1e9, d))
    dist = jnp.concatenate(parts, axis=1)
    lane = jax.lax.broadcasted_iota(jnp.int32, (P, chunk), 1)

    t = topk_ref[:, K - 1:K]
    below = dist < t
    cand_ref[...] = jnp.where(below, dist, jnp.inf)
    cnt = jnp.sum(jnp.where(below, 1.0, 0.0), axis=1)
    cmax = jnp.minimum(jnp.max(cnt), float(K))

    for i in range(K):
        @pl.when(i < cmax)
        def _extract():
            cv = cand_ref[...]
            m = jnp.min(cv, axis=1, keepdims=True)
            tk = topk_ref[...]
            shifted = jnp.concatenate(
                [jnp.full((P, 1), -jnp.inf, jnp.float32), tk[:, :K - 1]], axis=1)
            topk_ref[...] = jnp.minimum(jnp.maximum(shifted, m), tk)
            sel = jnp.min(jnp.where(cv == m, lane, chunk), axis=1, keepdims=True)
            cand_ref[...] = jnp.where(lane == sel, jnp.inf, cv)

    @pl.when(step == nsteps - 1)
    def _finish():
        out_ref[0] = topk_ref[...]


def _merge_body(tk_ref, out_ref):
    v = tk_ref[...]
    lane = jax.lax.broadcasted_iota(jnp.int32, (P, 2 * K), 1)
    acc = jnp.zeros((P, 1), jnp.float32)
    for _ in range(K):
        m = jnp.min(v, axis=1, keepdims=True)
        acc = acc + m
        sel = jnp.min(jnp.where(v == m, lane, 2 * K), axis=1, keepdims=True)
        v = jnp.where(lane == sel, jnp.inf, v)
    out_ref[...] = -jnp.sum(acc, axis=(0, 1), keepdims=True) / (P * K)


def kernel(feats1, feats2, positive_pairs):
    b, c, h, w = feats1.shape
    n = h * w
    f2_3d = feats2.reshape(c, h, w)
    neg_idx = _neg_indices(n)
    f1flat = feats1.reshape(c * n)
    ag = _make_sc_gather(c, n)(f1flat, neg_idx.astype(jnp.int32))
    anchors = ag.reshape(2, 16, c // 2, 16).transpose(1, 3, 0, 2).reshape(P, c)
    idx2d = neg_idx.reshape(P, 1).astype(jnp.int32)

    rows_per_step = 8
    chunk = rows_per_step * w
    nsteps_half = n // chunk // 2
    body = functools.partial(_dist_topk_body, chunk=chunk, nsteps=nsteps_half, w=w)
    tk2 = pl.pallas_call(
        body,
        grid=(2, nsteps_half),
        in_specs=[
            pl.BlockSpec((P, 1), lambda hh, i: (0, 0)),
            pl.BlockSpec((P, c), lambda hh, i: (0, 0)),
            pl.BlockSpec((c, rows_per_step, w),
                         lambda hh, i: (0, hh * nsteps_half + i, 0)),
        ],
        out_specs=pl.BlockSpec((1, P, K), lambda hh, i: (hh, 0, 0)),
        out_shape=jax.ShapeDtypeStruct((2, P, K), jnp.float32),
        scratch_shapes=[
            pltpu.VMEM((P, K), jnp.float32),
            pltpu.VMEM((P, chunk), jnp.float32),
        ],
        compiler_params=pltpu.CompilerParams(
            dimension_semantics=("parallel", "arbitrary")),
    )(idx2d, anchors, f2_3d)
    tk16 = tk2.transpose(1, 0, 2).reshape(P, 2 * K)
    out = pl.pallas_call(
        _merge_body,
        out_shape=jax.ShapeDtypeStruct((1, 1), jnp.float32),
    )(tk16)
    return out[0, 0]


# speedup vs baseline: 1.0900x; 1.0900x over previous
import functools

import jax
import jax.numpy as jnp
from jax import lax
from jax.experimental import pallas as pl
from jax.experimental.pallas import tpu as pltpu
from jax.experimental.pallas import tpu_sc as plsc

P = 256
K = 8
LIM = 5

_neg_idx_cache = {}


def _neg_indices(n):
    if n not in _neg_idx_cache:
        kn = jax.random.key(42)
        _neg_idx_cache[n] = jax.random.choice(kn, n, shape=(P,), replace=False)
    return _neg_idx_cache[n]


def _make_sc_gather(cf, n):
    chalf = cf // 2
    mesh = plsc.VectorSubcoreMesh(core_axis_name="c", subcore_axis_name="s")

    @functools.partial(
        pl.kernel, mesh=mesh,
        out_type=jax.ShapeDtypeStruct((2, 16, chalf * 16), jnp.float32),
        scratch_types=[
            pltpu.VMEM((16,), jnp.int32),
            pltpu.VMEM((chalf * 16,), jnp.int32),
            pltpu.VMEM((chalf * 16,), jnp.float32),
            pltpu.SemaphoreType.DMA,
        ],
    )
    def _gather(f1flat_hbm, idx_hbm, out_hbm, idx_v, flatidx_v, vals_v, sem):
        g = lax.axis_index("s")
        co = lax.axis_index("c")
        pltpu.sync_copy(idx_hbm.at[pl.ds(g * 16, 16)], idx_v)
        idx16 = idx_v[...]
        for cc in range(chalf):
            flatidx_v[pl.ds(cc * 16, 16)] = (co * chalf + cc) * n + idx16
        pltpu.async_copy(f1flat_hbm.at[flatidx_v], vals_v, sem).wait()
        pltpu.sync_copy(vals_v, out_hbm.at[co, g])

    return _gather


def _dist_topk_body(idx_ref, a_ref, f2_ref, out_ref, topk_ref, cand_ref,
                    *, chunk, nsteps, w):
    step = pl.program_id(0)

    @pl.when(step == 0)
    def _init():
        topk_ref[...] = jnp.full((P, K), jnp.inf, jnp.float32)

    a = a_ref[...]
    a2 = jnp.sum(a * a, axis=1, keepdims=True)
    rows = chunk // w
    idx = idx_ref[...]

    t = topk_ref[:, K - 1:K]
    cnt = jnp.zeros((P, 1), jnp.float32)
    for r in range(rows):
        f2r = f2_ref[:, r, :]
        b2 = jnp.sum(f2r * f2r, axis=0, keepdims=True)
        ab = jnp.dot(a, f2r, preferred_element_type=jnp.float32,
                     precision=jax.lax.Precision.DEFAULT)
        d = jax.nn.relu(a2 + b2 - 2.0 * ab)
        j = (step * chunk + r * w
             + jax.lax.broadcasted_iota(jnp.int32, (P, w), 1))
        q = j - idx + (w * LIM + LIM)
        excl = (j > 0) & (q >= 0) & (q < w * 2 * LIM) & ((q & (w - 1)) < 2 * LIM)
        d = jnp.where(excl, d + 1e9, d)
        below = d < t
        cand_ref[:, r * w:(r + 1) * w] = jnp.where(below, d, jnp.inf)
        cnt = cnt + jnp.sum(jnp.where(below, 1.0, 0.0), axis=1, keepdims=True)
    lane = jax.lax.broadcasted_iota(jnp.int32, (P, chunk), 1)
    cmax = jnp.minimum(jnp.max(cnt), float(K))

    for i in range(K):
        @pl.when(i < cmax)
        def _extract():
            cv = cand_ref[...]
            m = jnp.min(cv, axis=1, keepdims=True)
            tk = topk_ref[...]
            shifted = jnp.concatenate(
                [jnp.full((P, 1), -jnp.inf, jnp.float32), tk[:, :K - 1]], axis=1)
            topk_ref[...] = jnp.minimum(jnp.maximum(shifted, m), tk)
            sel = jnp.min(jnp.where(cv == m, lane, chunk), axis=1, keepdims=True)
            cand_ref[...] = jnp.where(lane == sel, jnp.inf, cv)

    @pl.when(step == nsteps - 1)
    def _finish():
        out_ref[...] = -jnp.sum(topk_ref[...], axis=(0, 1), keepdims=True) / (P * K)


def kernel(feats1, feats2, positive_pairs):
    b, c, h, w = feats1.shape
    n = h * w
    f2_3d = feats2.reshape(c, h, w)
    neg_idx = _neg_indices(n)
    f1flat = feats1.reshape(c * n)
    ag = _make_sc_gather(c, n)(f1flat, neg_idx.astype(jnp.int32))
    anchors = ag.reshape(2, 16, c // 2, 16).transpose(1, 3, 0, 2).reshape(P, c)
    idx2d = neg_idx.reshape(P, 1).astype(jnp.int32)

    rows_per_step = 8
    chunk = rows_per_step * w
    nsteps = n // chunk
    body = functools.partial(_dist_topk_body, chunk=chunk, nsteps=nsteps, w=w)
    out = pl.pallas_call(
        body,
        grid=(nsteps,),
        in_specs=[
            pl.BlockSpec((P, 1), lambda i: (0, 0)),
            pl.BlockSpec((P, c), lambda i: (0, 0)),
            pl.BlockSpec((c, rows_per_step, w), lambda i: (0, i, 0)),
        ],
        out_specs=pl.BlockSpec((1, 1), lambda i: (0, 0)),
        out_shape=jax.ShapeDtypeStruct((1, 1), jnp.float32),
        scratch_shapes=[
            pltpu.VMEM((P, K), jnp.float32),
            pltpu.VMEM((P, chunk), jnp.float32),
        ],
    )(idx2d, anchors, f2_3d)
    return out[0, 0]
